# baseline (speedup 1.0000x reference)
import functools

import jax
import jax.numpy as jnp
from jax import lax
from jax.experimental import pallas as pl
from jax.experimental.pallas import tpu as pltpu
from jax.experimental.pallas import tpu_sc as plsc

V = 100000
D = 128
B = 16384

_NC = 2
_NS = 16
_NW = _NC * _NS
_BPW = B // _NW
_CH = 64
_NCHUNK = _BPW // _CH
_L = 16


def _body(u_idx_hbm, v_idx_hbm, r_idx_hbm, eh_hbm, rvh_hbm, out_hbm,
          u_i, v_i, r_i, ua, va, ra, ub, vb, rb, out_v, sem_a, sem_b):
    wid = lax.axis_index("s") * _NC + lax.axis_index("c")
    base = wid * _BPW

    ci = (pltpu.async_copy(u_idx_hbm.at[pl.ds(base, _BPW)], u_i, sem_a),
          pltpu.async_copy(v_idx_hbm.at[pl.ds(base, _BPW)], v_i, sem_a),
          pltpu.async_copy(r_idx_hbm.at[pl.ds(base, _BPW)], r_i, sem_a))
    zero = jnp.zeros((_L,), jnp.float32)
    for j in range(_BPW // _L):
        out_v[pl.ds(j * _L, _L)] = zero
    for cp in ci:
        cp.wait()

    bufs = [(ua, va, ra, sem_a), (ub, vb, rb, sem_b)]

    def fire(c):
        u_b, v_b, r_b, sem = bufs[c % 2]
        sl = pl.ds(c * _CH, _CH)
        return (pltpu.async_copy(eh_hbm.at[u_i.at[sl]], u_b, sem),
                pltpu.async_copy(eh_hbm.at[v_i.at[sl]], v_b, sem),
                pltpu.async_copy(rvh_hbm.at[r_i.at[sl]], r_b, sem))

    mask15 = lax.iota(jnp.int32, _L) == (_L - 1)
    pending = fire(0)
    for c in range(_NCHUNK):
        nxt = fire(c + 1) if c + 1 < _NCHUNK else None
        for cp in pending:
            cp.wait()
        pending = nxt
        u_b, v_b, r_b, _ = bufs[c % 2]

        def row_fn(rr, _, c=c, u_b=u_b, v_b=v_b, r_b=r_b):
            for k in range(2):
                r = rr * 2 + k

                def term(d, r=r):
                    s = pl.ds(d * _L, _L)
                    return u_b[r, s] * v_b[r, s] * r_b[r, s]

                t = [term(d) for d in range(D // _L)]
                acc = (((t[0] + t[1]) + (t[2] + t[3]))
                       + ((t[4] + t[5]) + (t[6] + t[7])))
                pos = jnp.full((_L,), c * _CH + r, jnp.int32)
                plsc.addupdate_scatter(out_v, [pos], plsc.cumsum(acc),
                                       mask=mask15)
            return 0

        lax.fori_loop(0, _CH // 2, row_fn, 0)

    for j in range(_BPW // _L):
        out_v[pl.ds(j * _L, _L)] = -out_v[pl.ds(j * _L, _L)]
    pltpu.sync_copy(out_v, out_hbm.at[pl.ds(base, _BPW)])


@jax.jit
def kernel(u_idx, r_idx, v_idx, Eh, rvh):
    k = functools.partial(
        pl.kernel,
        out_type=jax.ShapeDtypeStruct((B,), jnp.float32),
        mesh=plsc.VectorSubcoreMesh(core_axis_name="c", subcore_axis_name="s"),
        scratch_types=[
            pltpu.VMEM((_BPW,), jnp.int32),
            pltpu.VMEM((_BPW,), jnp.int32),
            pltpu.VMEM((_BPW,), jnp.int32),
            pltpu.VMEM((_CH, D), jnp.float32),
            pltpu.VMEM((_CH, D), jnp.float32),
            pltpu.VMEM((_CH, D), jnp.float32),
            pltpu.VMEM((_CH, D), jnp.float32),
            pltpu.VMEM((_CH, D), jnp.float32),
            pltpu.VMEM((_CH, D), jnp.float32),
            pltpu.VMEM((_BPW,), jnp.float32),
            pltpu.SemaphoreType.DMA,
            pltpu.SemaphoreType.DMA,
        ],
        compiler_params=pltpu.CompilerParams(needs_layout_passes=False),
    )(_body)
    return k(u_idx.astype(jnp.int32), v_idx.astype(jnp.int32),
             r_idx.astype(jnp.int32), Eh, rvh)


# speedup vs baseline: 1.0391x; 1.0391x over previous
import functools

import jax
import jax.numpy as jnp
from jax import lax
from jax.experimental import pallas as pl
from jax.experimental.pallas import tpu as pltpu
from jax.experimental.pallas import tpu_sc as plsc

V = 100000
D = 128
B = 16384

_NC = 2
_NS = 16
_NW = _NC * _NS
_BPW = B // _NW
_CH = 64
_NCHUNK = _BPW // _CH
_L = 16


def _body(u_idx_hbm, v_idx_hbm, r_idx_hbm, eh_hbm, rvh_hbm, out_hbm,
          u_i, v_i, r_i, ua, va, ra, ub, vb, rb, out_v, sem_a, sem_b):
    wid = lax.axis_index("s") * _NC + lax.axis_index("c")
    base = wid * _BPW

    ci = (pltpu.async_copy(u_idx_hbm.at[pl.ds(base, _BPW)], u_i, sem_a),
          pltpu.async_copy(v_idx_hbm.at[pl.ds(base, _BPW)], v_i, sem_a),
          pltpu.async_copy(r_idx_hbm.at[pl.ds(base, _BPW)], r_i, sem_a))
    zero = jnp.zeros((_L,), jnp.float32)
    for j in range(_BPW // _L):
        out_v[pl.ds(j * _L, _L)] = zero
    for cp in ci:
        cp.wait()

    bufs = [(ua, va, ra, sem_a), (ub, vb, rb, sem_b)]

    def fire(c):
        u_b, v_b, r_b, sem = bufs[c % 2]
        sl = pl.ds(c * _CH, _CH)
        return (pltpu.async_copy(eh_hbm.at[u_i.at[sl]], u_b, sem),
                pltpu.async_copy(eh_hbm.at[v_i.at[sl]], v_b, sem),
                pltpu.async_copy(rvh_hbm.at[r_i.at[sl]], r_b, sem))

    mask15 = lax.iota(jnp.int32, _L) == (_L - 1)
    pending = fire(0)
    for c in range(_NCHUNK):
        nxt = fire(c + 1) if c + 1 < _NCHUNK else None
        for cp in pending:
            cp.wait()
        pending = nxt
        u_b, v_b, r_b, _ = bufs[c % 2]

        def row_fn(r, _, c=c, u_b=u_b, v_b=v_b, r_b=r_b):
            def term(d):
                s = pl.ds(d * _L, _L)
                return u_b[r, s] * v_b[r, s] * r_b[r, s]

            t = [term(d) for d in range(D // _L)]
            acc = (((t[0] + t[1]) + (t[2] + t[3]))
                   + ((t[4] + t[5]) + (t[6] + t[7])))
            pos = jnp.full((_L,), c * _CH + r, jnp.int32)
            plsc.addupdate_scatter(out_v, [pos], plsc.cumsum(acc), mask=mask15)
            return 0

        lax.fori_loop(0, _CH, row_fn, 0)

    for j in range(_BPW // _L):
        out_v[pl.ds(j * _L, _L)] = -out_v[pl.ds(j * _L, _L)]
    pltpu.sync_copy(out_v, out_hbm.at[pl.ds(base, _BPW)])


@jax.jit
def kernel(u_idx, r_idx, v_idx, Eh, rvh):
    k = functools.partial(
        pl.kernel,
        out_type=jax.ShapeDtypeStruct((B,), jnp.float32),
        mesh=plsc.VectorSubcoreMesh(core_axis_name="c", subcore_axis_name="s"),
        scratch_types=[
            pltpu.VMEM((_BPW,), jnp.int32),
            pltpu.VMEM((_BPW,), jnp.int32),
            pltpu.VMEM((_BPW,), jnp.int32),
            pltpu.VMEM((_CH, D), jnp.float32),
            pltpu.VMEM((_CH, D), jnp.float32),
            pltpu.VMEM((_CH, D), jnp.float32),
            pltpu.VMEM((_CH, D), jnp.float32),
            pltpu.VMEM((_CH, D), jnp.float32),
            pltpu.VMEM((_CH, D), jnp.float32),
            pltpu.VMEM((_BPW,), jnp.float32),
            pltpu.SemaphoreType.DMA,
            pltpu.SemaphoreType.DMA,
        ],
        compiler_params=pltpu.CompilerParams(needs_layout_passes=False),
    )(_body)
    return k(u_idx.astype(jnp.int32), v_idx.astype(jnp.int32),
             r_idx.astype(jnp.int32), Eh, rvh)
